# Initial kernel scaffold; baseline (speedup 1.0000x reference)
import jax
import jax.numpy as jnp
from jax.experimental import pallas as pl


def kernel(x, bm_0, b_0, gamma_0, beta_0, bm_1, b_1, gamma_1, beta_1, bm_2, b_2, gamma_2, beta_2, bm_3, b_3, gamma_3, beta_3, bm_4, b_4):
    raise NotImplementedError("write your pallas kernel here")


# speedup vs baseline: 13.6322x; 13.6322x over previous
import functools

import jax
import jax.numpy as jnp
from jax import lax
from jax.experimental import pallas as pl
from jax.experimental.pallas import tpu as pltpu

_VMEM_LIMIT = 48 * 1024 * 1024


def _layer_kernel(*refs, act, B, H, W, Cin, C):
    if act:
        x_ref, wr0_ref, wr1_ref, sc_ref, sh_ref, out_ref, mom_ref = refs
        xf = x_ref[...].astype(jnp.float32)
        xf = xf * sc_ref[...].reshape(1, 1, 1, Cin) + sh_ref[...].reshape(1, 1, 1, Cin)
        xb = jnp.maximum(xf, 0.0).astype(jnp.bfloat16)
    else:
        x_ref, wr0_ref, wr1_ref, out_ref, mom_ref = refs
        xb = x_ref[...]

    zrow = jnp.zeros((B, 1, W, Cin), jnp.bfloat16)
    xm = jnp.concatenate([zrow, xb[:, :-1]], axis=1)
    xp = jnp.concatenate([xb[:, 1:], zrow], axis=1)

    zcol = jnp.zeros((B, H, 1, C), jnp.float32)
    sums = []
    sqs = []
    for r, (sec, w_ref) in enumerate(((xm, wr0_ref), (xp, wr1_ref))):
        a = jnp.concatenate([xb, sec], axis=-1).reshape(B * H * W, 2 * Cin)
        t = jnp.dot(a, w_ref[...], preferred_element_type=jnp.float32)
        t = t.reshape(B, H, W, 4 * C)
        t0 = t[..., 0 * C:1 * C]
        t1 = t[..., 1 * C:2 * C]
        t2 = t[..., 2 * C:3 * C]
        t3 = t[..., 3 * C:4 * C]
        e = t1 + jnp.concatenate([zcol, t3[:, :, :-1]], axis=2)
        o = t2 + jnp.concatenate([t0[:, :, 1:], zcol], axis=2)
        y = jnp.concatenate([e, o], axis=-1).astype(jnp.bfloat16)
        out_ref[:, :, r, :, :] = y
        yf = y.astype(jnp.float32).reshape(B * H * W, 2 * C)
        s2 = jnp.sum(yf, axis=0, keepdims=True)
        q2 = jnp.sum(yf * yf, axis=0, keepdims=True)
        sums.append(s2[:, :C] + s2[:, C:])
        sqs.append(q2[:, :C] + q2[:, C:])
    mom_ref[0, 0, :] = (sums[0] + sums[1]).reshape(C)
    mom_ref[0, 1, :] = (sqs[0] + sqs[1]).reshape(C)


def _conv_layer(x, wr0, wr1, scale, shift, B):
    N, H, W, Cin = x.shape
    C = wr0.shape[1] // 4
    G = N // B
    act = scale is not None

    in_specs = [
        pl.BlockSpec((B, H, W, Cin), lambda i: (i, 0, 0, 0)),
        pl.BlockSpec((2 * Cin, 4 * C), lambda i: (0, 0)),
        pl.BlockSpec((2 * Cin, 4 * C), lambda i: (0, 0)),
    ]
    args = [x, wr0, wr1]
    if act:
        in_specs += [
            pl.BlockSpec((1, Cin), lambda i: (0, 0)),
            pl.BlockSpec((1, Cin), lambda i: (0, 0)),
        ]
        args += [scale.reshape(1, Cin), shift.reshape(1, Cin)]

    y5, mom = pl.pallas_call(
        functools.partial(_layer_kernel, act=act, B=B, H=H, W=W, Cin=Cin, C=C),
        out_shape=(
            jax.ShapeDtypeStruct((N, H, 2, W, 2 * C), jnp.bfloat16),
            jax.ShapeDtypeStruct((G, 2, C), jnp.float32),
        ),
        grid_spec=pltpu.PrefetchScalarGridSpec(
            num_scalar_prefetch=0,
            grid=(G,),
            in_specs=in_specs,
            out_specs=(
                pl.BlockSpec((B, H, 2, W, 2 * C), lambda i: (i, 0, 0, 0, 0)),
                pl.BlockSpec((1, 2, C), lambda i: (i, 0, 0)),
            ),
        ),
        compiler_params=pltpu.CompilerParams(
            dimension_semantics=("parallel",),
            vmem_limit_bytes=_VMEM_LIMIT,
        ),
    )(*args)
    return y5, mom


def _final_kernel(x_ref, w_ref, b_ref, sc_ref, sh_ref, out_ref, *, P, W):
    cin = sc_ref.shape[1]
    xf = x_ref[0].astype(jnp.float32)
    xf = xf * sc_ref[...].reshape(1, cin) + sh_ref[...].reshape(1, cin)
    xb = jnp.maximum(xf, 0.0).astype(jnp.bfloat16)
    t = lax.dot_general(w_ref[...], xb, (((0,), (1,)), ((), ())),
                        preferred_element_type=jnp.float32)
    t = t.reshape(16, 3, P)

    jj = lax.broadcasted_iota(jnp.int32, (3, P), 1) % W
    first = jj == 0
    last = jj == W - 1
    z1 = jnp.zeros((3, 1), jnp.float32)
    zW = jnp.zeros((3, W), jnp.float32)

    ecol = []
    ocol = []
    for kh in range(4):
        t0 = t[4 * kh + 0]
        t1 = t[4 * kh + 1]
        t2 = t[4 * kh + 2]
        t3 = t[4 * kh + 3]
        sh_m = jnp.concatenate([z1, t3[:, :-1]], axis=1)
        ecol.append(t1 + jnp.where(first, 0.0, sh_m))
        sh_p = jnp.concatenate([t0[:, 1:], z1], axis=1)
        ocol.append(t2 + jnp.where(last, 0.0, sh_p))

    bias = b_ref[...]
    for s, col in enumerate((ecol, ocol)):
        rm = jnp.concatenate([zW, col[3][:, :-W]], axis=1)
        rp = jnp.concatenate([col[0][:, W:], zW], axis=1)
        out_ref[0, 0 + s] = jnp.tanh(col[1] + rm + bias)
        out_ref[0, 2 + s] = jnp.tanh(col[2] + rp + bias)


def _final_layer(x, bm4, b4, scale, shift):
    N, P, Cin = x.shape
    W = 64
    out = pl.pallas_call(
        functools.partial(_final_kernel, P=P, W=W),
        out_shape=jax.ShapeDtypeStruct((N, 4, 3, P), jnp.float32),
        grid_spec=pltpu.PrefetchScalarGridSpec(
            num_scalar_prefetch=0,
            grid=(N,),
            in_specs=[
                pl.BlockSpec((1, P, Cin), lambda i: (i, 0, 0)),
                pl.BlockSpec((Cin, 48), lambda i: (0, 0)),
                pl.BlockSpec((3, 1), lambda i: (0, 0)),
                pl.BlockSpec((1, Cin), lambda i: (0, 0)),
                pl.BlockSpec((1, Cin), lambda i: (0, 0)),
            ],
            out_specs=pl.BlockSpec((1, 4, 3, P), lambda i: (i, 0, 0, 0)),
        ),
        compiler_params=pltpu.CompilerParams(
            dimension_semantics=("parallel",),
            vmem_limit_bytes=_VMEM_LIMIT,
        ),
    )(x, bm4, b4.reshape(3, 1).astype(jnp.float32),
      scale.reshape(1, Cin), shift.reshape(1, Cin))
    return out


def _split_weights(bm):
    cin = bm.shape[0]
    c4 = bm.shape[1] // 4
    w = [bm[:, k * c4:(k + 1) * c4] for k in range(4)]
    wr0 = jnp.concatenate([w[1], w[3]], axis=0)
    wr1 = jnp.concatenate([w[2], w[0]], axis=0)
    return wr0, wr1


def _bn_stats(mom, count, gamma, beta):
    tot = jnp.sum(mom.astype(jnp.float32), axis=0)
    mean = tot[0] / count
    msq = tot[1] / count
    var = jnp.maximum(msq - jnp.square(mean), 0.0)
    scale = gamma * lax.rsqrt(var + 1e-5)
    shift = beta - mean * scale
    return scale, shift


def _pick_b(n, pref):
    b = min(pref, n)
    while n % b:
        b -= 1
    return b


@jax.jit
def _forward(x, params):
    N = x.shape[0]
    xb = jnp.transpose(x, (0, 2, 3, 1)).astype(jnp.bfloat16)

    scale = shift = None
    prefs = (32, 8, 4, 2)
    for li in range(4):
        bm, gamma, beta = params[li]
        wr0, wr1 = _split_weights(bm)
        B = _pick_b(N, prefs[li])
        y5, mom = _conv_layer(xb, wr0, wr1, scale, shift, B)
        Nn, H, _, W, C2 = y5.shape
        C = C2 // 2
        count = jnp.float32(Nn * 2 * H * 2 * W)
        scale, shift = _bn_stats(mom, count, gamma, beta)
        xb = y5.reshape(Nn, 2 * H, 2 * W, C)

    bm4, b4 = params[4]
    Nn, H, W, Cin = xb.shape
    xf = xb.reshape(Nn, H * W, Cin)
    o5 = _final_layer(xf, bm4, b4, scale, shift)
    o = o5.reshape(Nn, 2, 2, 3, H, W)
    o = jnp.transpose(o, (0, 3, 4, 1, 5, 2))
    return o.reshape(Nn, 3, 2 * H, 2 * W)


def kernel(x, bm_0, b_0, gamma_0, beta_0, bm_1, b_1, gamma_1, beta_1,
           bm_2, b_2, gamma_2, beta_2, bm_3, b_3, gamma_3, beta_3,
           bm_4, b_4):
    params = [
        (bm_0, gamma_0, beta_0),
        (bm_1, gamma_1, beta_1),
        (bm_2, gamma_2, beta_2),
        (bm_3, gamma_3, beta_3),
        (bm_4, b_4),
    ]
    return _forward(x, params)
